# baseline (speedup 1.0000x reference)
import functools

import jax
import jax.numpy as jnp
from jax import lax
from jax.experimental import pallas as pl
from jax.experimental.pallas import tpu as pltpu
from jax.experimental.pallas import tpu_sc as plsc

B = 8
N = 2048
NC = 2
NS = 16
NW = NC * NS
ROWS_PER_W = N // NW
GROUPS = ROWS_PER_W // 16
UNROLL = 16
NBLK = N // UNROLL


def _sc_chamfer_body(xt_hbm, yt_hbm, out_hbm,
                     x0, x1, x2, y0, y1, y2, c2x, c2y, res, sem):
    wid = lax.axis_index("s") * NC + lax.axis_index("c")
    row_base = wid * ROWS_PER_W

    def make_prep_c2(r0, r1, r2, c2):
        def prep_c2(i, carry):
            sl = pl.ds(i * 16, 16)
            v0, v1, v2 = r0[sl], r1[sl], r2[sl]
            c2[sl] = -0.5 * (v0 * v0 + v1 * v1 + v2 * v2)
            return carry
        return prep_c2

    def one_direction(rows, pts, c2, ctx):
        r0, r1, r2 = rows
        p0, p1, p2 = pts
        total = jnp.zeros((16,), jnp.float32)
        for g in range(GROUPS):
            sl = pl.ds(row_base + g * 16, 16)
            xr0, xr1, xr2 = r0[sl], r1[sl], r2[sl]

            def scan_blocks(j, carry):
                bv, bi = carry
                msl = pl.ds(j * UNROLL, UNROLL)
                pv0, pv1, pv2, cv = p0[msl], p1[msl], p2[msl], c2[msl]
                ts = []
                for u in range(UNROLL):
                    s0 = pv0[u]
                    s1 = pv1[u]
                    s2 = pv2[u]
                    cu = cv[u]
                    ts.append(xr0 * s0 + (xr1 * s1 + (xr2 * s2 + cu)))
                while len(ts) > 1:
                    ts = [jnp.maximum(ts[2 * k], ts[2 * k + 1])
                          for k in range(len(ts) // 2)]
                blk = ts[0]
                mask = blk > bv
                bv = jnp.where(mask, blk, bv)
                bi = jnp.where(mask, jnp.zeros((16,), jnp.int32) + j, bi)
                return bv, bi

            bv0 = jnp.full((16,), -jnp.inf, jnp.float32)
            bi0 = jnp.zeros((16,), jnp.int32)
            bv, bi = lax.fori_loop(0, NBLK, scan_blocks, (bv0, bi0))

            base_idx = bi * UNROLL
            bv2 = jnp.full((16,), -jnp.inf, jnp.float32)
            mi = jnp.zeros((16,), jnp.int32)
            for u in range(UNROLL):
                idx = base_idx + u
                g0 = plsc.load_gather(p0, [idx])
                g1 = plsc.load_gather(p1, [idx])
                g2 = plsc.load_gather(p2, [idx])
                gc = plsc.load_gather(c2, [idx])
                t = xr0 * g0 + (xr1 * g1 + (xr2 * g2 + gc))
                mask = t > bv2
                bv2 = jnp.where(mask, t, bv2)
                mi = jnp.where(mask, idx, mi)

            w0 = plsc.load_gather(p0, [mi])
            w1 = plsc.load_gather(p1, [mi])
            w2 = plsc.load_gather(p2, [mi])
            f5 = jnp.zeros((16,), jnp.float32)
            for xr, w in ((xr0, w0), (xr1, w1), (xr2, w2)):
                a = jnp.abs(xr - w)
                a2 = a * a
                f5 = f5 + a2 * a2 * a
            total = total + f5
        res[ctx, :] = total

    def per_batch(b, carry):
        pltpu.sync_copy(xt_hbm.at[b * 3 + 0], x0)
        pltpu.sync_copy(xt_hbm.at[b * 3 + 1], x1)
        pltpu.sync_copy(xt_hbm.at[b * 3 + 2], x2)
        pltpu.sync_copy(yt_hbm.at[b * 3 + 0], y0)
        pltpu.sync_copy(yt_hbm.at[b * 3 + 1], y1)
        pltpu.sync_copy(yt_hbm.at[b * 3 + 2], y2)
        lax.fori_loop(0, N // 16, make_prep_c2(x0, x1, x2, c2x), 0)
        lax.fori_loop(0, N // 16, make_prep_c2(y0, y1, y2, c2y), 0)
        one_direction((x0, x1, x2), (y0, y1, y2), c2y, b)
        one_direction((y0, y1, y2), (x0, x1, x2), c2x, b + B)
        return carry

    lax.fori_loop(0, B, per_batch, 0)
    pltpu.sync_copy(res, out_hbm.at[wid])


def _epilogue_body(parts_ref, out_ref):
    s = jnp.sum(parts_ref[...], axis=1, keepdims=True)
    out_ref[...] = jnp.sum(s ** 0.2, axis=(0, 1), keepdims=True) * (1.0 / B)


@jax.jit
def kernel(x, y):
    xt = jnp.transpose(x, (0, 2, 1)).reshape(B * 3, N)
    yt = jnp.transpose(y, (0, 2, 1)).reshape(B * 3, N)

    mesh = plsc.VectorSubcoreMesh(core_axis_name="c", subcore_axis_name="s",
                                  num_cores=NC, num_subcores=NS)
    sc_call = pl.kernel(
        _sc_chamfer_body,
        out_type=jax.ShapeDtypeStruct((NW, 2 * B, 16), jnp.float32),
        mesh=mesh,
        compiler_params=pltpu.CompilerParams(needs_layout_passes=False),
        scratch_types=[
            pltpu.VMEM((N,), jnp.float32),
            pltpu.VMEM((N,), jnp.float32),
            pltpu.VMEM((N,), jnp.float32),
            pltpu.VMEM((N,), jnp.float32),
            pltpu.VMEM((N,), jnp.float32),
            pltpu.VMEM((N,), jnp.float32),
            pltpu.VMEM((N,), jnp.float32),
            pltpu.VMEM((N,), jnp.float32),
            pltpu.VMEM((2 * B, 16), jnp.float32),
            pltpu.SemaphoreType.DMA,
        ],
    )
    parts = sc_call(xt, yt)

    parts2 = jnp.transpose(parts, (1, 0, 2)).reshape(2 * B, NW * 16)
    out = pl.pallas_call(
        _epilogue_body,
        out_shape=jax.ShapeDtypeStruct((1, 1), jnp.float32),
    )(parts2)
    return out[0, 0]


# speedup vs baseline: 2.4698x; 2.4698x over previous
import jax
import jax.numpy as jnp
from jax import lax
from jax.experimental import pallas as pl
from jax.experimental.pallas import tpu as pltpu
from jax.experimental.pallas import tpu_sc as plsc

B = 8
N = 2048
MC = 512
NMC = N // MC
BIG = 2**30

NC = 2
NS = 16
NW = NC * NS
ROWS_PER_W = N // NW
GROUPS = ROWS_PER_W // 16


def _tc_score_body(x_ref, y_ref, o1_ref, o2_ref):
    mc = pl.program_id(1)
    xb = x_ref[0]
    yb = y_ref[0]
    xc = x_ref[0, pl.ds(mc * MC, MC), :]
    yc = y_ref[0, pl.ds(mc * MC, MC), :]
    nxb = jnp.sum(xb * xb, axis=1, keepdims=True)
    nyb = jnp.sum(yb * yb, axis=1, keepdims=True)
    rows = lax.broadcasted_iota(jnp.int32, (N, MC), 0)

    s1 = lax.dot_general(yb, xc, (((1,), (1,)), ((), ())),
                         preferred_element_type=jnp.float32)
    sc1 = s1 - 0.5 * nyb
    m1 = jnp.max(sc1, axis=0, keepdims=True)
    o1_ref[...] = jnp.min(jnp.where(sc1 == m1, rows, BIG), axis=0,
                          keepdims=True)[None, None]

    s2 = lax.dot_general(xb, yc, (((1,), (1,)), ((), ())),
                         preferred_element_type=jnp.float32)
    sc2 = s2 - 0.5 * nxb
    m2 = jnp.max(sc2, axis=0, keepdims=True)
    o2_ref[...] = jnp.min(jnp.where(sc2 == m2, rows, BIG), axis=0,
                          keepdims=True)[None, None]


def _sc_gather_body(xt_hbm, yt_hbm, d1_hbm, d2_hbm, out_hbm,
                    x0, x1, x2, y0, y1, y2, i1, i2, res, sem):
    wid = lax.axis_index("s") * NC + lax.axis_index("c")
    row_base = wid * ROWS_PER_W

    def one_direction(rows, pts, idx, ctx):
        r0, r1, r2 = rows
        p0, p1, p2 = pts
        total = jnp.zeros((16,), jnp.float32)
        for g in range(GROUPS):
            sl = pl.ds(row_base + g * 16, 16)
            iv = idx[sl]
            w0 = plsc.load_gather(p0, [iv])
            w1 = plsc.load_gather(p1, [iv])
            w2 = plsc.load_gather(p2, [iv])
            f5 = jnp.zeros((16,), jnp.float32)
            for xr, w in ((r0[sl], w0), (r1[sl], w1), (r2[sl], w2)):
                a = jnp.abs(xr - w)
                a2 = a * a
                f5 = f5 + a2 * a2 * a
            total = total + f5
        res[ctx, :] = total

    def per_batch(b, carry):
        pltpu.sync_copy(xt_hbm.at[b * 3 + 0], x0)
        pltpu.sync_copy(xt_hbm.at[b * 3 + 1], x1)
        pltpu.sync_copy(xt_hbm.at[b * 3 + 2], x2)
        pltpu.sync_copy(yt_hbm.at[b * 3 + 0], y0)
        pltpu.sync_copy(yt_hbm.at[b * 3 + 1], y1)
        pltpu.sync_copy(yt_hbm.at[b * 3 + 2], y2)
        pltpu.sync_copy(d1_hbm.at[b], i1)
        pltpu.sync_copy(d2_hbm.at[b], i2)
        one_direction((x0, x1, x2), (y0, y1, y2), i1, b)
        one_direction((y0, y1, y2), (x0, x1, x2), i2, b + B)
        return carry

    lax.fori_loop(0, B, per_batch, 0)
    pltpu.sync_copy(res, out_hbm.at[wid])


def _epilogue_body(parts_ref, out_ref):
    s = jnp.sum(parts_ref[...], axis=1, keepdims=True)
    out_ref[...] = jnp.sum(s ** 0.2, axis=(0, 1), keepdims=True) * (1.0 / B)


@jax.jit
def kernel(x, y):
    d1, d2 = pl.pallas_call(
        _tc_score_body,
        grid=(B, NMC),
        in_specs=[
            pl.BlockSpec((1, N, 3), lambda b, mc: (b, 0, 0)),
            pl.BlockSpec((1, N, 3), lambda b, mc: (b, 0, 0)),
        ],
        out_specs=[
            pl.BlockSpec((1, 1, 1, MC), lambda b, mc: (b, mc, 0, 0)),
            pl.BlockSpec((1, 1, 1, MC), lambda b, mc: (b, mc, 0, 0)),
        ],
        out_shape=[
            jax.ShapeDtypeStruct((B, NMC, 1, MC), jnp.int32),
            jax.ShapeDtypeStruct((B, NMC, 1, MC), jnp.int32),
        ],
        compiler_params=pltpu.CompilerParams(
            dimension_semantics=("arbitrary", "arbitrary")),
    )(x, y)
    d1 = d1.reshape(B, N)
    d2 = d2.reshape(B, N)

    xt = jnp.transpose(x, (0, 2, 1)).reshape(B * 3, N)
    yt = jnp.transpose(y, (0, 2, 1)).reshape(B * 3, N)

    mesh = plsc.VectorSubcoreMesh(core_axis_name="c", subcore_axis_name="s",
                                  num_cores=NC, num_subcores=NS)
    sc_call = pl.kernel(
        _sc_gather_body,
        out_type=jax.ShapeDtypeStruct((NW, 2 * B, 16), jnp.float32),
        mesh=mesh,
        compiler_params=pltpu.CompilerParams(needs_layout_passes=False),
        scratch_types=[
            pltpu.VMEM((N,), jnp.float32),
            pltpu.VMEM((N,), jnp.float32),
            pltpu.VMEM((N,), jnp.float32),
            pltpu.VMEM((N,), jnp.float32),
            pltpu.VMEM((N,), jnp.float32),
            pltpu.VMEM((N,), jnp.float32),
            pltpu.VMEM((N,), jnp.int32),
            pltpu.VMEM((N,), jnp.int32),
            pltpu.VMEM((2 * B, 16), jnp.float32),
            pltpu.SemaphoreType.DMA,
        ],
    )
    parts = sc_call(xt, yt, d1, d2)

    parts2 = jnp.transpose(parts, (1, 0, 2)).reshape(2 * B, NW * 16)
    out = pl.pallas_call(
        _epilogue_body,
        out_shape=jax.ShapeDtypeStruct((1, 1), jnp.float32),
    )(parts2)
    return out[0, 0]


# speedup vs baseline: 3.9311x; 1.5917x over previous
import jax
import jax.numpy as jnp
from jax import lax
from jax.experimental import pallas as pl
from jax.experimental.pallas import tpu as pltpu
from jax.experimental.pallas import tpu_sc as plsc

B = 8
N = 2048
MC = 512
NMC = N // MC
BIG = 2**30

NC = 2
NS = 16
NW = NC * NS
ROWS_PER_W = N // NW
GROUPS = ROWS_PER_W // 16


def _tc_score_body(x_ref, y_ref, o1_ref, o2_ref):
    mc = pl.program_id(1)
    xb = x_ref[0]
    yb = y_ref[0]
    xc = x_ref[0, pl.ds(mc * MC, MC), :]
    yc = y_ref[0, pl.ds(mc * MC, MC), :]
    nxb = jnp.sum(xb * xb, axis=1, keepdims=True)
    nyb = jnp.sum(yb * yb, axis=1, keepdims=True)
    ones_c = jnp.ones((MC, 1), jnp.float32)

    ya = jnp.concatenate([yb, -0.5 * nyb], axis=1)
    xc1 = jnp.concatenate([xc, ones_c], axis=1)
    sc1 = lax.dot_general(ya, xc1, (((1,), (1,)), ((), ())),
                          preferred_element_type=jnp.float32)
    o1_ref[...] = jnp.argmax(sc1, axis=0).astype(jnp.int32).reshape(1, 1, 1, MC)

    xa = jnp.concatenate([xb, -0.5 * nxb], axis=1)
    yc1 = jnp.concatenate([yc, ones_c], axis=1)
    sc2 = lax.dot_general(xa, yc1, (((1,), (1,)), ((), ())),
                          preferred_element_type=jnp.float32)
    o2_ref[...] = jnp.argmax(sc2, axis=0).astype(jnp.int32).reshape(1, 1, 1, MC)


def _sc_gather_body(xf_hbm, yf_hbm, d1_hbm, d2_hbm, out_hbm,
                    xs, ys, i1, i2, res, sem):
    wid = lax.axis_index("s") * NC + lax.axis_index("c")
    row_base = wid * ROWS_PER_W

    cp = [
        pltpu.async_copy(xf_hbm, xs, sem),
        pltpu.async_copy(yf_hbm, ys, sem),
    ]
    for b in range(B):
        src = pl.ds(b * N + row_base, ROWS_PER_W)
        dst = pl.ds(b * ROWS_PER_W, ROWS_PER_W)
        cp.append(pltpu.async_copy(d1_hbm.at[src], i1.at[dst], sem))
        cp.append(pltpu.async_copy(d2_hbm.at[src], i2.at[dst], sem))
    for c in cp:
        c.wait()

    lane3 = lax.iota(jnp.int32, 16) * 3

    def one_direction(rows_flat, pts_flat, idx, b, ctx):
        pbase = b * (3 * N)
        total = jnp.zeros((16,), jnp.float32)
        for g in range(GROUPS):
            iv = idx[pl.ds(b * ROWS_PER_W + g * 16, 16)]
            wflat = pbase + iv * 3
            rflat = pbase + (row_base + g * 16) * 3 + lane3
            f5 = jnp.zeros((16,), jnp.float32)
            for d in range(3):
                w = plsc.load_gather(pts_flat, [wflat + d])
                r = plsc.load_gather(rows_flat, [rflat + d])
                a = jnp.abs(r - w)
                a2 = a * a
                f5 = f5 + a2 * a2 * a
            total = total + f5
        res[ctx, :] = total

    def per_batch(b, carry):
        one_direction(xs, ys, i1, b, b)
        one_direction(ys, xs, i2, b, b + B)
        return carry

    lax.fori_loop(0, B, per_batch, 0)
    pltpu.sync_copy(res, out_hbm.at[wid])


def _epilogue_body(parts_ref, out_ref):
    s = jnp.sum(parts_ref[...], axis=1, keepdims=True)
    out_ref[...] = jnp.sum(s ** 0.2, axis=(0, 1), keepdims=True) * (1.0 / B)


@jax.jit
def kernel(x, y):
    d1, d2 = pl.pallas_call(
        _tc_score_body,
        grid=(B, NMC),
        in_specs=[
            pl.BlockSpec((1, N, 3), lambda b, mc: (b, 0, 0)),
            pl.BlockSpec((1, N, 3), lambda b, mc: (b, 0, 0)),
        ],
        out_specs=[
            pl.BlockSpec((1, 1, 1, MC), lambda b, mc: (b, mc, 0, 0)),
            pl.BlockSpec((1, 1, 1, MC), lambda b, mc: (b, mc, 0, 0)),
        ],
        out_shape=[
            jax.ShapeDtypeStruct((B, NMC, 1, MC), jnp.int32),
            jax.ShapeDtypeStruct((B, NMC, 1, MC), jnp.int32),
        ],
        compiler_params=pltpu.CompilerParams(
            dimension_semantics=("arbitrary", "arbitrary")),
    )(x, y)
    d1 = d1.reshape(B * N)
    d2 = d2.reshape(B * N)

    xf = x.reshape(B * N * 3)
    yf = y.reshape(B * N * 3)

    mesh = plsc.VectorSubcoreMesh(core_axis_name="c", subcore_axis_name="s",
                                  num_cores=NC, num_subcores=NS)
    sc_call = pl.kernel(
        _sc_gather_body,
        out_type=jax.ShapeDtypeStruct((NW, 2 * B, 16), jnp.float32),
        mesh=mesh,
        compiler_params=pltpu.CompilerParams(needs_layout_passes=False),
        scratch_types=[
            pltpu.VMEM((B * N * 3,), jnp.float32),
            pltpu.VMEM((B * N * 3,), jnp.float32),
            pltpu.VMEM((B * ROWS_PER_W,), jnp.int32),
            pltpu.VMEM((B * ROWS_PER_W,), jnp.int32),
            pltpu.VMEM((2 * B, 16), jnp.float32),
            pltpu.SemaphoreType.DMA,
        ],
    )
    parts = sc_call(xf, yf, d1, d2)

    parts2 = jnp.transpose(parts, (1, 0, 2)).reshape(2 * B, NW * 16)
    out = pl.pallas_call(
        _epilogue_body,
        out_shape=jax.ShapeDtypeStruct((1, 1), jnp.float32),
    )(parts2)
    return out[0, 0]


# speedup vs baseline: 3.9430x; 1.0030x over previous
import jax
import jax.numpy as jnp
from jax import lax
from jax.experimental import pallas as pl
from jax.experimental.pallas import tpu as pltpu
from jax.experimental.pallas import tpu_sc as plsc

B = 8
N = 2048
MC = 512
NMC = N // MC
BIG = 2**30

NC = 2
NS = 16
NW = NC * NS
ROWS_PER_W = N // NW
GROUPS = ROWS_PER_W // 16


def _tc_score_body(x_ref, y_ref, o1_ref, o2_ref):
    mc = pl.program_id(1)
    xb = x_ref[0]
    yb = y_ref[0]
    xc = x_ref[0, pl.ds(mc * MC, MC), :]
    yc = y_ref[0, pl.ds(mc * MC, MC), :]
    nxb = jnp.sum(xb * xb, axis=1, keepdims=True)
    nyb = jnp.sum(yb * yb, axis=1, keepdims=True)
    ones_c = jnp.ones((MC, 1), jnp.float32)

    ya = jnp.concatenate([yb, -0.5 * nyb], axis=1)
    xc1 = jnp.concatenate([xc, ones_c], axis=1)
    sc1 = lax.dot_general(ya, xc1, (((1,), (1,)), ((), ())),
                          preferred_element_type=jnp.float32)
    o1_ref[...] = jnp.argmax(sc1, axis=0).astype(jnp.int32).reshape(1, 1, 1, MC)

    xa = jnp.concatenate([xb, -0.5 * nxb], axis=1)
    yc1 = jnp.concatenate([yc, ones_c], axis=1)
    sc2 = lax.dot_general(xa, yc1, (((1,), (1,)), ((), ())),
                          preferred_element_type=jnp.float32)
    o2_ref[...] = jnp.argmax(sc2, axis=0).astype(jnp.int32).reshape(1, 1, 1, MC)


def _sc_gather_body(xf_hbm, yf_hbm, d1_hbm, d2_hbm, out_hbm,
                    xs, ys, i1, i2, res, sem):
    wid = lax.axis_index("s") * NC + lax.axis_index("c")
    row_base = wid * ROWS_PER_W

    mcw = row_base // MC
    off = row_base % MC
    cp = [
        pltpu.async_copy(xf_hbm, xs, sem),
        pltpu.async_copy(yf_hbm, ys, sem),
    ]
    for b in range(B):
        src = (b, mcw, 0, pl.ds(off, ROWS_PER_W))
        dst = pl.ds(b * ROWS_PER_W, ROWS_PER_W)
        cp.append(pltpu.async_copy(d1_hbm.at[src], i1.at[dst], sem))
        cp.append(pltpu.async_copy(d2_hbm.at[src], i2.at[dst], sem))
    for c in cp:
        c.wait()

    lane3 = lax.iota(jnp.int32, 16) * 3

    def one_direction(rows_flat, pts_flat, idx, b, ctx):
        pbase = b * (3 * N)
        total = jnp.zeros((16,), jnp.float32)
        for g in range(GROUPS):
            iv = idx[pl.ds(b * ROWS_PER_W + g * 16, 16)]
            wflat = pbase + iv * 3
            rflat = pbase + (row_base + g * 16) * 3 + lane3
            f5 = jnp.zeros((16,), jnp.float32)
            for d in range(3):
                w = plsc.load_gather(pts_flat, [wflat + d])
                r = plsc.load_gather(rows_flat, [rflat + d])
                a = jnp.abs(r - w)
                a2 = a * a
                f5 = f5 + a2 * a2 * a
            total = total + f5
        res[ctx, :] = total

    def per_batch(b, carry):
        one_direction(xs, ys, i1, b, b)
        one_direction(ys, xs, i2, b, b + B)
        return carry

    lax.fori_loop(0, B, per_batch, 0)
    pltpu.sync_copy(res, out_hbm.at[wid])


def _epilogue_body(parts_ref, out_ref):
    s = jnp.sum(parts_ref[...], axis=1, keepdims=True)
    out_ref[...] = jnp.sum(s ** 0.2, axis=(0, 1), keepdims=True) * (1.0 / B)


@jax.jit
def kernel(x, y):
    d1, d2 = pl.pallas_call(
        _tc_score_body,
        grid=(B, NMC),
        in_specs=[
            pl.BlockSpec((1, N, 3), lambda b, mc: (b, 0, 0)),
            pl.BlockSpec((1, N, 3), lambda b, mc: (b, 0, 0)),
        ],
        out_specs=[
            pl.BlockSpec((1, 1, 1, MC), lambda b, mc: (b, mc, 0, 0)),
            pl.BlockSpec((1, 1, 1, MC), lambda b, mc: (b, mc, 0, 0)),
        ],
        out_shape=[
            jax.ShapeDtypeStruct((B, NMC, 1, MC), jnp.int32),
            jax.ShapeDtypeStruct((B, NMC, 1, MC), jnp.int32),
        ],
        compiler_params=pltpu.CompilerParams(
            dimension_semantics=("arbitrary", "arbitrary")),
    )(x, y)
    xf = x.reshape(B * N * 3)
    yf = y.reshape(B * N * 3)

    mesh = plsc.VectorSubcoreMesh(core_axis_name="c", subcore_axis_name="s",
                                  num_cores=NC, num_subcores=NS)
    sc_call = pl.kernel(
        _sc_gather_body,
        out_type=jax.ShapeDtypeStruct((NW, 2 * B, 16), jnp.float32),
        mesh=mesh,
        compiler_params=pltpu.CompilerParams(needs_layout_passes=False),
        scratch_types=[
            pltpu.VMEM((B * N * 3,), jnp.float32),
            pltpu.VMEM((B * N * 3,), jnp.float32),
            pltpu.VMEM((B * ROWS_PER_W,), jnp.int32),
            pltpu.VMEM((B * ROWS_PER_W,), jnp.int32),
            pltpu.VMEM((2 * B, 16), jnp.float32),
            pltpu.SemaphoreType.DMA,
        ],
    )
    parts = sc_call(xf, yf, d1, d2)

    parts2 = jnp.transpose(parts, (1, 0, 2)).reshape(2 * B, NW * 16)
    out = pl.pallas_call(
        _epilogue_body,
        out_shape=jax.ShapeDtypeStruct((1, 1), jnp.float32),
    )(parts2)
    return out[0, 0]


# speedup vs baseline: 4.2245x; 1.0714x over previous
import jax
import jax.numpy as jnp
from jax import lax
from jax.experimental import pallas as pl
from jax.experimental.pallas import tpu as pltpu
from jax.experimental.pallas import tpu_sc as plsc

B = 8
N = 2048
MC = 512
NMC = N // MC
BIG = 2**30

NC = 2
NS = 16
NW = NC * NS
ROWS_PER_W = N // NW
GROUPS = ROWS_PER_W // 16


def _tc_score_body(x_ref, y_ref, o1_ref, o2_ref):
    mc = pl.program_id(1)
    xb = x_ref[0]
    yb = y_ref[0]
    xc = x_ref[0, pl.ds(mc * MC, MC), :]
    yc = y_ref[0, pl.ds(mc * MC, MC), :]
    nxb = jnp.sum(xb * xb, axis=1, keepdims=True)
    nyb = jnp.sum(yb * yb, axis=1, keepdims=True)
    ones_c = jnp.ones((MC, 1), jnp.float32)

    ya = jnp.concatenate([yb, -0.5 * nyb], axis=1)
    xc1 = jnp.concatenate([xc, ones_c], axis=1)
    sc1 = lax.dot_general(ya, xc1, (((1,), (1,)), ((), ())),
                          preferred_element_type=jnp.float32)
    o1_ref[...] = jnp.argmax(sc1, axis=0).astype(jnp.int32).reshape(1, 1, 1, MC)

    xa = jnp.concatenate([xb, -0.5 * nxb], axis=1)
    yc1 = jnp.concatenate([yc, ones_c], axis=1)
    sc2 = lax.dot_general(xa, yc1, (((1,), (1,)), ((), ())),
                          preferred_element_type=jnp.float32)
    o2_ref[...] = jnp.argmax(sc2, axis=0).astype(jnp.int32).reshape(1, 1, 1, MC)


def _sc_gather_body(xyf_hbm, d1_hbm, d2_hbm, out_hbm,
                    xys, i1, i2, res, sem):
    wid = lax.axis_index("s") * NC + lax.axis_index("c")
    row_base = wid * ROWS_PER_W

    mcw = row_base // MC
    off = row_base % MC
    cp = [
        pltpu.async_copy(xyf_hbm, xys, sem),
    ]
    for b in range(B):
        src = (b, mcw, 0, pl.ds(off, ROWS_PER_W))
        dst = pl.ds(b * ROWS_PER_W, ROWS_PER_W)
        cp.append(pltpu.async_copy(d1_hbm.at[src], i1.at[dst], sem))
        cp.append(pltpu.async_copy(d2_hbm.at[src], i2.at[dst], sem))
    for c in cp:
        c.wait()

    lane3 = lax.iota(jnp.int32, 16) * 3

    def one_direction(rbase, pbase, idx, b, ctx):
        total = jnp.zeros((16,), jnp.float32)
        for g in range(GROUPS):
            iv = idx[pl.ds(b * ROWS_PER_W + g * 16, 16)]
            wflat = pbase + iv * 3
            rflat = rbase + (row_base + g * 16) * 3 + lane3
            f5 = jnp.zeros((16,), jnp.float32)
            for d in range(3):
                w = plsc.load_gather(xys, [wflat + d])
                r = plsc.load_gather(xys, [rflat + d])
                a = jnp.abs(r - w)
                a2 = a * a
                f5 = f5 + a2 * a2 * a
            total = total + f5
        res[ctx, :] = total

    def per_batch(b, carry):
        xb = b * (3 * N)
        yb = B * N * 3 + b * (3 * N)
        one_direction(xb, yb, i1, b, b)
        one_direction(yb, xb, i2, b, b + B)
        return carry

    lax.fori_loop(0, B, per_batch, 0)
    pltpu.sync_copy(res, out_hbm.at[wid])


def _epilogue_body(parts_ref, out_ref):
    s = jnp.sum(parts_ref[...], axis=1, keepdims=True)
    out_ref[...] = jnp.sum(s ** 0.2, axis=(0, 1), keepdims=True) * (1.0 / B)


@jax.jit
def kernel(x, y):
    d1, d2 = pl.pallas_call(
        _tc_score_body,
        grid=(B, NMC),
        in_specs=[
            pl.BlockSpec((1, N, 3), lambda b, mc: (b, 0, 0)),
            pl.BlockSpec((1, N, 3), lambda b, mc: (b, 0, 0)),
        ],
        out_specs=[
            pl.BlockSpec((1, 1, 1, MC), lambda b, mc: (b, mc, 0, 0)),
            pl.BlockSpec((1, 1, 1, MC), lambda b, mc: (b, mc, 0, 0)),
        ],
        out_shape=[
            jax.ShapeDtypeStruct((B, NMC, 1, MC), jnp.int32),
            jax.ShapeDtypeStruct((B, NMC, 1, MC), jnp.int32),
        ],
        compiler_params=pltpu.CompilerParams(
            dimension_semantics=("arbitrary", "arbitrary")),
    )(x, y)
    xyf = jnp.concatenate([x.reshape(B * N * 3), y.reshape(B * N * 3)])

    mesh = plsc.VectorSubcoreMesh(core_axis_name="c", subcore_axis_name="s",
                                  num_cores=NC, num_subcores=NS)
    sc_call = pl.kernel(
        _sc_gather_body,
        out_type=jax.ShapeDtypeStruct((NW, 2 * B, 16), jnp.float32),
        mesh=mesh,
        compiler_params=pltpu.CompilerParams(needs_layout_passes=False),
        scratch_types=[
            pltpu.VMEM((2 * B * N * 3,), jnp.float32),
            pltpu.VMEM((B * ROWS_PER_W,), jnp.int32),
            pltpu.VMEM((B * ROWS_PER_W,), jnp.int32),
            pltpu.VMEM((2 * B, 16), jnp.float32),
            pltpu.SemaphoreType.DMA,
        ],
    )
    parts = sc_call(xyf, d1, d2)

    parts2 = jnp.transpose(parts, (1, 0, 2)).reshape(2 * B, NW * 16)
    out = pl.pallas_call(
        _epilogue_body,
        out_shape=jax.ShapeDtypeStruct((1, 1), jnp.float32),
    )(parts2)
    return out[0, 0]
